# Initial kernel scaffold; baseline (speedup 1.0000x reference)
import functools

import jax
import jax.numpy as jnp
from jax import lax
from jax.experimental import pallas as pl
from jax.experimental.pallas import tpu as pltpu
from jax.experimental.pallas import tpu_sc as plsc

H = W = 2048
NPIX = H * W
NBINS = 128 * 128 * 128

_BR = 128


def _idx_body(r_ref, g_ref, b_ref, out_ref):
    r = r_ref[...] >> 1
    g = g_ref[...] >> 1
    b = b_ref[...] >> 1
    out_ref[...] = (r << 14) | (g << 7) | b


def _bin_index(r8, g8, b8):
    return pl.pallas_call(
        _idx_body,
        grid=(H // _BR,),
        in_specs=[pl.BlockSpec((_BR, W), lambda i: (i, 0))] * 3,
        out_specs=pl.BlockSpec((_BR, W), lambda i: (i, 0)),
        out_shape=jax.ShapeDtypeStruct((H, W), jnp.int32),
    )(r8, g8, b8)


_NTILES = 16
_CHUNK = 8192
_PER_TILE = NPIX // _NTILES
_NCHUNK = _PER_TILE // _CHUNK
_BIN_SLICE = NBINS // _NTILES

_sc_mesh = plsc.VectorSubcoreMesh(core_axis_name="c", subcore_axis_name="s")


@functools.partial(
    pl.kernel,
    mesh=_sc_mesh,
    out_type=[
        jax.ShapeDtypeStruct((NBINS,), jnp.float32),
        jax.ShapeDtypeStruct((NBINS,), jnp.float32),
    ],
    scratch_types=[
        pltpu.VMEM((_CHUNK,), jnp.int32),
        pltpu.VMEM((_CHUNK,), jnp.float32),
        pltpu.VMEM((_CHUNK,), jnp.float32),
        pltpu.VMEM_SHARED((NBINS,), jnp.float32),
    ],
)
def _hist_sc(idx_hbm, mask_hbm, full_out, lines_out, idx_v, val_v, z_v, hist_s):
    cid = lax.axis_index("c")
    sid = lax.axis_index("s")

    def _fill(buf, value):
        def body(i, _):
            buf[pl.ds(i * 16, 16)] = jnp.full((16,), value, buf.dtype)
            return 0
        lax.fori_loop(0, _CHUNK // 16, body, 0)

    _fill(z_v, 0.0)
    _fill(val_v, 1.0)

    def zbody(k, _):
        pltpu.sync_copy(z_v, hist_s.at[pl.ds(sid * _BIN_SLICE + k * _CHUNK, _CHUNK)])
        return 0
    lax.fori_loop(0, _BIN_SLICE // _CHUNK, zbody, 0)
    plsc.subcore_barrier()

    def body(i, _):
        base = sid * _PER_TILE + i * _CHUNK
        pltpu.sync_copy(idx_hbm.at[pl.ds(base, _CHUNK)], idx_v)

        @pl.when(cid == 1)
        def _():
            pltpu.sync_copy(mask_hbm.at[pl.ds(base, _CHUNK)], val_v)

        pltpu.sync_copy(val_v, hist_s.at[idx_v], add=True)
        return 0
    lax.fori_loop(0, _NCHUNK, body, 0)
    plsc.subcore_barrier()

    off = sid * _BIN_SLICE

    @pl.when(cid == 0)
    def _():
        pltpu.sync_copy(hist_s.at[pl.ds(off, _BIN_SLICE)],
                        full_out.at[pl.ds(off, _BIN_SLICE)])

    @pl.when(cid == 1)
    def _():
        pltpu.sync_copy(hist_s.at[pl.ds(off, _BIN_SLICE)],
                        lines_out.at[pl.ds(off, _BIN_SLICE)])


_RROWS = NBINS // 128
_RBLK = 2048
_RGRID = _RROWS // _RBLK


def _ratios_body(full_ref, lines_ref, newfull_ref, filt_ref, sl_sm, slw_sm):
    j = pl.program_id(0)
    i = pl.program_id(1)
    f = full_ref[...]
    l = lines_ref[...] + 1e-10
    w = jnp.log(l / (f + 1.0))
    newfull_ref[...] = f.astype(jnp.int32)

    @pl.when(j == 0)
    def _():
        @pl.when(i == 0)
        def _z():
            sl_sm[0] = 0.0
            slw_sm[0] = 0.0
        sl_sm[0] += jnp.sum(l)
        slw_sm[0] += jnp.sum(l * w)
        filt_ref[...] = jnp.zeros(w.shape, jnp.int32)

    @pl.when(j == 1)
    def _():
        c = slw_sm[0] / sl_sm[0]
        filt_ref[...] = (w > c).astype(jnp.int32)


def _ratios(full_f32, lines_f32):
    f2 = full_f32.reshape(_RROWS, 128)
    l2 = lines_f32.reshape(_RROWS, 128)
    return pl.pallas_call(
        _ratios_body,
        grid=(2, _RGRID),
        in_specs=[pl.BlockSpec((_RBLK, 128), lambda j, i: (i, 0))] * 2,
        out_specs=[pl.BlockSpec((_RBLK, 128), lambda j, i: (i, 0))] * 2,
        out_shape=[
            jax.ShapeDtypeStruct((_RROWS, 128), jnp.int32),
            jax.ShapeDtypeStruct((_RROWS, 128), jnp.int32),
        ],
        scratch_shapes=[pltpu.SMEM((1,), jnp.float32)] * 2,
    )(f2, l2)


def kernel(img, mask, full, lines):
    r8 = img[:, :, 0]
    g8 = img[:, :, 1]
    b8 = img[:, :, 2]
    idx = _bin_index(r8, g8, b8).reshape(NPIX)
    full_f32, lines_f32 = _hist_sc(idx, mask.reshape(NPIX))
    new_full, filt = _ratios(full_f32, lines_f32)
    return (new_full.reshape(128, 128, 128),
            lines_f32.reshape(128, 128, 128),
            filt.astype(jnp.bool_).reshape(128, 128, 128))


# speedup vs baseline: 19.3063x; 19.3063x over previous
import functools

import jax
import jax.numpy as jnp
from jax import lax
from jax.experimental import pallas as pl
from jax.experimental.pallas import tpu as pltpu
from jax.experimental.pallas import tpu_sc as plsc

H = W = 2048
NPIX = H * W
NBINS = 128 * 128 * 128
HALF = NBINS // 2
TRASH = 2048

_BR = 256


def _idx_body(r_ref, g_ref, b_ref, lo_ref, hi_ref):
    r = r_ref[...] >> 1
    g = g_ref[...] >> 1
    b = b_ref[...] >> 1
    lin = (r << 14) | (g << 7) | b
    trash = HALF + (lax.broadcasted_iota(jnp.int32, lin.shape, 1) & (TRASH - 1))
    in_lo = lin < HALF
    lo_ref[...] = jnp.where(in_lo, lin, trash)
    hi_ref[...] = jnp.where(in_lo, trash, lin - HALF)


def _bin_index(r8, g8, b8):
    return pl.pallas_call(
        _idx_body,
        grid=(H // _BR,),
        in_specs=[pl.BlockSpec((_BR, W), lambda i: (i, 0))] * 3,
        out_specs=[pl.BlockSpec((_BR, W), lambda i: (i, 0))] * 2,
        out_shape=[jax.ShapeDtypeStruct((H, W), jnp.int32)] * 2,
    )(r8, g8, b8)


_NTILES = 16
_CHUNK = 8192
_PER_TILE = NPIX // _NTILES
_NCHUNK = _PER_TILE // _CHUNK
_HWORDS = HALF + TRASH
_BIN_SLICE = HALF // _NTILES


@functools.cache
def _make_hist_sc():
    mesh = plsc.VectorSubcoreMesh(core_axis_name="c", subcore_axis_name="s")
    return functools.partial(
        pl.kernel,
        mesh=mesh,
        out_type=[
            jax.ShapeDtypeStruct((NBINS,), jnp.float32),
            jax.ShapeDtypeStruct((NBINS,), jnp.float32),
        ],
        scratch_types=[
            pltpu.VMEM((_CHUNK,), jnp.int32),
            pltpu.VMEM((_CHUNK,), jnp.float32),
            pltpu.VMEM((_CHUNK,), jnp.float32),
            pltpu.VMEM_SHARED((_HWORDS,), jnp.float32),
        ],
    )(_hist_sc_body)


def _hist_sc_body(lo_hbm, hi_hbm, mask_hbm, full_out, lines_out,
                  idx_v, val_v, z_v, hist_s):
    cid = lax.axis_index("c")
    sid = lax.axis_index("s")

    def _fill(buf, value):
        def body(i, _):
            buf[pl.ds(i * 16, 16)] = jnp.full((16,), value, buf.dtype)
            return 0
        lax.fori_loop(0, _CHUNK // 16, body, 0)

    _fill(z_v, 0.0)
    _fill(val_v, 1.0)

    for half, idx_hbm in ((0, lo_hbm), (1, hi_hbm)):
        def zbody(k, _):
            pltpu.sync_copy(z_v, hist_s.at[pl.ds(sid * _BIN_SLICE + k * _CHUNK, _CHUNK)])
            return 0
        lax.fori_loop(0, _BIN_SLICE // _CHUNK, zbody, 0)
        plsc.subcore_barrier()

        def body(i, _):
            base = sid * _PER_TILE + i * _CHUNK
            pltpu.sync_copy(idx_hbm.at[pl.ds(base, _CHUNK)], idx_v)

            @pl.when(cid == 1)
            def _():
                pltpu.sync_copy(mask_hbm.at[pl.ds(base, _CHUNK)], val_v)

            pltpu.sync_copy(val_v, hist_s.at[idx_v], add=True)
            return 0
        lax.fori_loop(0, _NCHUNK, body, 0)
        plsc.subcore_barrier()

        src = sid * _BIN_SLICE
        dst = half * HALF + src

        @pl.when(cid == 0)
        def _():
            pltpu.sync_copy(hist_s.at[pl.ds(src, _BIN_SLICE)],
                            full_out.at[pl.ds(dst, _BIN_SLICE)])

        @pl.when(cid == 1)
        def _():
            pltpu.sync_copy(hist_s.at[pl.ds(src, _BIN_SLICE)],
                            lines_out.at[pl.ds(dst, _BIN_SLICE)])


_RROWS = NBINS // 128
_RBLK = 2048
_RGRID = _RROWS // _RBLK


def _ratios_body(full_ref, lines_ref, newfull_ref, filt_ref, sl_sm, slw_sm):
    j = pl.program_id(0)
    i = pl.program_id(1)
    f = full_ref[...]
    l = lines_ref[...] + 1e-10
    w = jnp.log(l / (f + 1.0))
    newfull_ref[...] = f.astype(jnp.int32)

    @pl.when(j == 0)
    def _():
        @pl.when(i == 0)
        def _z():
            sl_sm[0] = 0.0
            slw_sm[0] = 0.0
        sl_sm[0] += jnp.sum(l)
        slw_sm[0] += jnp.sum(l * w)
        filt_ref[...] = jnp.zeros(w.shape, jnp.int32)

    @pl.when(j == 1)
    def _():
        c = slw_sm[0] / sl_sm[0]
        filt_ref[...] = (w > c).astype(jnp.int32)


def _ratios(full_f32, lines_f32):
    f2 = full_f32.reshape(_RROWS, 128)
    l2 = lines_f32.reshape(_RROWS, 128)
    return pl.pallas_call(
        _ratios_body,
        grid=(2, _RGRID),
        in_specs=[pl.BlockSpec((_RBLK, 128), lambda j, i: (i, 0))] * 2,
        out_specs=[pl.BlockSpec((_RBLK, 128), lambda j, i: (i, 0))] * 2,
        out_shape=[
            jax.ShapeDtypeStruct((_RROWS, 128), jnp.int32),
            jax.ShapeDtypeStruct((_RROWS, 128), jnp.int32),
        ],
        scratch_shapes=[pltpu.SMEM((1,), jnp.float32)] * 2,
    )(f2, l2)


def kernel(img, mask, full, lines):
    r8 = img[:, :, 0]
    g8 = img[:, :, 1]
    b8 = img[:, :, 2]
    idx_lo, idx_hi = _bin_index(r8, g8, b8)
    full_f32, lines_f32 = _make_hist_sc()(
        idx_lo.reshape(NPIX), idx_hi.reshape(NPIX), mask.reshape(NPIX))
    new_full, filt = _ratios(full_f32, lines_f32)
    return (new_full.reshape(128, 128, 128),
            lines_f32.reshape(128, 128, 128),
            filt.astype(jnp.bool_).reshape(128, 128, 128))


# speedup vs baseline: 25.8932x; 1.3412x over previous
import functools

import jax
import jax.numpy as jnp
from jax import lax
from jax.experimental import pallas as pl
from jax.experimental.pallas import tpu as pltpu
from jax.experimental.pallas import tpu_sc as plsc

H = W = 2048
NPIX = H * W
NBINS = 128 * 128 * 128
HALF = NBINS // 2
TRASH = 2048

_BR = 256


def _idx_body(r_ref, g_ref, b_ref, lo_ref, hi_ref):
    r = r_ref[...] >> 1
    g = g_ref[...] >> 1
    b = b_ref[...] >> 1
    lin = (r << 14) | (g << 7) | b
    trash = HALF + (lax.broadcasted_iota(jnp.int32, lin.shape, 1) & (TRASH - 1))
    in_lo = lin < HALF
    lo_ref[...] = jnp.where(in_lo, lin, trash)
    hi_ref[...] = jnp.where(in_lo, trash, lin - HALF)


def _bin_index(r8, g8, b8):
    return pl.pallas_call(
        _idx_body,
        grid=(H // _BR,),
        in_specs=[pl.BlockSpec((_BR, W), lambda i: (i, 0))] * 3,
        out_specs=[pl.BlockSpec((_BR, W), lambda i: (i, 0))] * 2,
        out_shape=[jax.ShapeDtypeStruct((H, W), jnp.int32)] * 2,
    )(r8, g8, b8)


_NTILES = 16
_CHUNK = 8192
_PER_TILE = NPIX // _NTILES
_NCHUNK = _PER_TILE // _CHUNK
_ZCHUNK = 8192
_HWORDS = HALF + TRASH
_BIN_SLICE = HALF // _NTILES


@functools.cache
def _make_hist_sc():
    mesh = plsc.VectorSubcoreMesh(core_axis_name="c", subcore_axis_name="s")
    return functools.partial(
        pl.kernel,
        mesh=mesh,
        out_type=[
            jax.ShapeDtypeStruct((NBINS,), jnp.float32),
            jax.ShapeDtypeStruct((NBINS,), jnp.float32),
        ],
        scratch_types=[
            pltpu.VMEM((_CHUNK,), jnp.int32),
            pltpu.VMEM((_CHUNK,), jnp.int32),
            pltpu.VMEM((_CHUNK,), jnp.float32),
            pltpu.VMEM((_CHUNK,), jnp.float32),
            pltpu.VMEM((_ZCHUNK,), jnp.float32),
            pltpu.VMEM_SHARED((_HWORDS,), jnp.float32),
            pltpu.SemaphoreType.DMA,
            pltpu.SemaphoreType.DMA,
            pltpu.SemaphoreType.DMA,
            pltpu.SemaphoreType.DMA,
        ],
    )(_hist_sc_body)


def _hist_sc_body(lo_hbm, hi_hbm, mask_hbm, full_out, lines_out,
                  idx_v0, idx_v1, val_v0, val_v1, z_v, hist_s,
                  isem0, isem1, vsem0, vsem1):
    cid = lax.axis_index("c")
    sid = lax.axis_index("s")
    idx_bufs = (idx_v0, idx_v1)
    val_bufs = (val_v0, val_v1)
    isems = (isem0, isem1)
    vsems = (vsem0, vsem1)

    def _fill(buf, value):
        def body(i, _):
            buf[pl.ds(i * 16, 16)] = jnp.full((16,), value, buf.dtype)
            return 0
        lax.fori_loop(0, buf.shape[0] // 16, body, 0)

    _fill(z_v, 0.0)
    _fill(val_v0, 1.0)
    _fill(val_v1, 1.0)

    def start_chunk(idx_hbm, g, b):
        base = sid * _PER_TILE + g * _CHUNK
        pltpu.make_async_copy(idx_hbm.at[pl.ds(base, _CHUNK)],
                              idx_bufs[b], isems[b]).start()

        @pl.when(cid == 1)
        def _():
            pltpu.make_async_copy(mask_hbm.at[pl.ds(base, _CHUNK)],
                                  val_bufs[b], vsems[b]).start()

    def wait_chunk(b):
        pltpu.make_async_copy(lo_hbm.at[pl.ds(0, _CHUNK)],
                              idx_bufs[b], isems[b]).wait()

        @pl.when(cid == 1)
        def _():
            pltpu.make_async_copy(mask_hbm.at[pl.ds(0, _CHUNK)],
                                  val_bufs[b], vsems[b]).wait()

    for half, idx_hbm in ((0, lo_hbm), (1, hi_hbm)):
        start_chunk(idx_hbm, 0, 0)
        start_chunk(idx_hbm, 1, 1)

        def zbody(k, _):
            pltpu.sync_copy(z_v, hist_s.at[pl.ds(sid * _BIN_SLICE + k * _ZCHUNK, _ZCHUNK)])
            return 0
        lax.fori_loop(0, _BIN_SLICE // _ZCHUNK, zbody, 0)
        plsc.subcore_barrier()

        def pair(gg, _):
            for b in range(2):
                g = 2 * gg + b
                wait_chunk(b)
                pltpu.sync_copy(val_bufs[b], hist_s.at[idx_bufs[b]], add=True)

                @pl.when(g + 2 < _NCHUNK)
                def _():
                    start_chunk(idx_hbm, g + 2, b)
            return 0
        lax.fori_loop(0, _NCHUNK // 2, pair, 0)
        plsc.subcore_barrier()

        src = sid * _BIN_SLICE
        dst = half * HALF + src

        @pl.when(cid == 0)
        def _():
            pltpu.sync_copy(hist_s.at[pl.ds(src, _BIN_SLICE)],
                            full_out.at[pl.ds(dst, _BIN_SLICE)])

        @pl.when(cid == 1)
        def _():
            pltpu.sync_copy(hist_s.at[pl.ds(src, _BIN_SLICE)],
                            lines_out.at[pl.ds(dst, _BIN_SLICE)])


_RROWS = NBINS // 128
_RBLK = 2048
_RGRID = _RROWS // _RBLK


def _ratios_body(full_ref, lines_ref, newfull_ref, filt_ref, sl_sm, slw_sm):
    j = pl.program_id(0)
    i = pl.program_id(1)
    f = full_ref[...]
    l = lines_ref[...] + 1e-10
    w = jnp.log(l / (f + 1.0))
    newfull_ref[...] = f.astype(jnp.int32)

    @pl.when(j == 0)
    def _():
        @pl.when(i == 0)
        def _z():
            sl_sm[0] = 0.0
            slw_sm[0] = 0.0
        sl_sm[0] += jnp.sum(l)
        slw_sm[0] += jnp.sum(l * w)
        filt_ref[...] = jnp.zeros(w.shape, jnp.int32)

    @pl.when(j == 1)
    def _():
        c = slw_sm[0] / sl_sm[0]
        filt_ref[...] = (w > c).astype(jnp.int32)


def _ratios(full_f32, lines_f32):
    f2 = full_f32.reshape(_RROWS, 128)
    l2 = lines_f32.reshape(_RROWS, 128)
    return pl.pallas_call(
        _ratios_body,
        grid=(2, _RGRID),
        in_specs=[pl.BlockSpec((_RBLK, 128), lambda j, i: (i, 0))] * 2,
        out_specs=[pl.BlockSpec((_RBLK, 128), lambda j, i: (i, 0))] * 2,
        out_shape=[
            jax.ShapeDtypeStruct((_RROWS, 128), jnp.int32),
            jax.ShapeDtypeStruct((_RROWS, 128), jnp.int32),
        ],
        scratch_shapes=[pltpu.SMEM((1,), jnp.float32)] * 2,
    )(f2, l2)


def kernel(img, mask, full, lines):
    r8 = img[:, :, 0]
    g8 = img[:, :, 1]
    b8 = img[:, :, 2]
    idx_lo, idx_hi = _bin_index(r8, g8, b8)
    full_f32, lines_f32 = _make_hist_sc()(
        idx_lo.reshape(NPIX), idx_hi.reshape(NPIX), mask.reshape(NPIX))
    new_full, filt = _ratios(full_f32, lines_f32)
    return (new_full.reshape(128, 128, 128),
            lines_f32.reshape(128, 128, 128),
            filt.astype(jnp.bool_).reshape(128, 128, 128))


# speedup vs baseline: 29.6364x; 1.1446x over previous
import functools

import jax
import jax.numpy as jnp
from jax import lax
from jax.experimental import pallas as pl
from jax.experimental.pallas import tpu as pltpu
from jax.experimental.pallas import tpu_sc as plsc

H = W = 2048
NPIX = H * W
NBINS = 128 * 128 * 128
HALF = NBINS // 2
TRASH = 2048

_CB = W // 128


def _idx_body(r_ref, g_ref, b_ref, m_ref, lo_ref, hi_ref, mp_ref):
    r = r_ref[...] >> 1
    g = g_ref[...] >> 1
    b = b_ref[...] >> 1
    lin = (r << 14) | (g << 7) | b
    trash = HALF + (lax.broadcasted_iota(jnp.int32, lin.shape, 1) & (TRASH - 1))
    in_lo = lin < HALF
    lo_ref[0] = jnp.where(in_lo, lin, trash)
    hi_ref[0] = jnp.where(in_lo, trash, lin - HALF)
    mp_ref[0] = m_ref[...]


def _bin_index(r8, g8, b8, mask):
    return pl.pallas_call(
        _idx_body,
        grid=(_CB,),
        in_specs=[pl.BlockSpec((H, 128), lambda i: (0, i))] * 4,
        out_specs=[pl.BlockSpec((1, H, 128), lambda i: (i, 0, 0))] * 3,
        out_shape=[jax.ShapeDtypeStruct((_CB, H, 128), jnp.int32)] * 2
        + [jax.ShapeDtypeStruct((_CB, H, 128), jnp.float32)],
    )(r8, g8, b8, mask)


_NTILES = 16
_CHUNK = 8192
_PER_TILE = NPIX // _NTILES
_NCHUNK = _PER_TILE // _CHUNK
_ZCHUNK = 8192
_HWORDS = HALF + TRASH
_BIN_SLICE = HALF // _NTILES


@functools.cache
def _make_hist_sc():
    mesh = plsc.VectorSubcoreMesh(core_axis_name="c", subcore_axis_name="s")
    return functools.partial(
        pl.kernel,
        mesh=mesh,
        out_type=[
            jax.ShapeDtypeStruct((NBINS,), jnp.float32),
            jax.ShapeDtypeStruct((NBINS,), jnp.float32),
        ],
        scratch_types=[
            pltpu.VMEM((_CHUNK,), jnp.int32),
            pltpu.VMEM((_CHUNK,), jnp.int32),
            pltpu.VMEM((_CHUNK,), jnp.float32),
            pltpu.VMEM((_CHUNK,), jnp.float32),
            pltpu.VMEM((_ZCHUNK,), jnp.float32),
            pltpu.VMEM_SHARED((_HWORDS,), jnp.float32),
            pltpu.SemaphoreType.DMA,
            pltpu.SemaphoreType.DMA,
            pltpu.SemaphoreType.DMA,
            pltpu.SemaphoreType.DMA,
        ],
    )(_hist_sc_body)


def _hist_sc_body(lo_hbm, hi_hbm, mask_hbm, full_out, lines_out,
                  idx_v0, idx_v1, val_v0, val_v1, z_v, hist_s,
                  isem0, isem1, vsem0, vsem1):
    cid = lax.axis_index("c")
    sid = lax.axis_index("s")
    idx_bufs = (idx_v0, idx_v1)
    val_bufs = (val_v0, val_v1)
    isems = (isem0, isem1)
    vsems = (vsem0, vsem1)

    def _fill(buf, value):
        def body(i, _):
            buf[pl.ds(i * 16, 16)] = jnp.full((16,), value, buf.dtype)
            return 0
        lax.fori_loop(0, buf.shape[0] // 16, body, 0)

    _fill(z_v, 0.0)
    _fill(val_v0, 1.0)
    _fill(val_v1, 1.0)

    def start_chunk(idx_hbm, g, b):
        base = sid * _PER_TILE + g * _CHUNK
        pltpu.make_async_copy(idx_hbm.at[pl.ds(base, _CHUNK)],
                              idx_bufs[b], isems[b]).start()

        @pl.when(cid == 1)
        def _():
            pltpu.make_async_copy(mask_hbm.at[pl.ds(base, _CHUNK)],
                                  val_bufs[b], vsems[b]).start()

    def wait_chunk(b):
        pltpu.make_async_copy(lo_hbm.at[pl.ds(0, _CHUNK)],
                              idx_bufs[b], isems[b]).wait()

        @pl.when(cid == 1)
        def _():
            pltpu.make_async_copy(mask_hbm.at[pl.ds(0, _CHUNK)],
                                  val_bufs[b], vsems[b]).wait()

    for half, idx_hbm in ((0, lo_hbm), (1, hi_hbm)):
        start_chunk(idx_hbm, 0, 0)
        start_chunk(idx_hbm, 1, 1)

        def zbody(k, _):
            pltpu.sync_copy(z_v, hist_s.at[pl.ds(sid * _BIN_SLICE + k * _ZCHUNK, _ZCHUNK)])
            return 0
        lax.fori_loop(0, _BIN_SLICE // _ZCHUNK, zbody, 0)
        plsc.subcore_barrier()

        def pair(gg, _):
            for b in range(2):
                g = 2 * gg + b
                wait_chunk(b)
                pltpu.sync_copy(val_bufs[b], hist_s.at[idx_bufs[b]], add=True)

                @pl.when(g + 2 < _NCHUNK)
                def _():
                    start_chunk(idx_hbm, g + 2, b)
            return 0
        lax.fori_loop(0, _NCHUNK // 2, pair, 0)
        plsc.subcore_barrier()

        src = sid * _BIN_SLICE
        dst = half * HALF + src

        @pl.when(cid == 0)
        def _():
            pltpu.sync_copy(hist_s.at[pl.ds(src, _BIN_SLICE)],
                            full_out.at[pl.ds(dst, _BIN_SLICE)])

        @pl.when(cid == 1)
        def _():
            pltpu.sync_copy(hist_s.at[pl.ds(src, _BIN_SLICE)],
                            lines_out.at[pl.ds(dst, _BIN_SLICE)])


_RROWS = NBINS // 128
_RBLK = 2048
_RGRID = _RROWS // _RBLK


def _sums_body(full_ref, lines_ref, newfull_ref, sl_ref, slw_ref):
    i = pl.program_id(0)
    f = full_ref[...]
    l = lines_ref[...] + 1e-10
    w = jnp.log(l / (f + 1.0))
    newfull_ref[...] = f.astype(jnp.int32)

    @pl.when(i == 0)
    def _z():
        sl_ref[0, 0] = 0.0
        slw_ref[0, 0] = 0.0
    sl_ref[0, 0] += jnp.sum(l)
    slw_ref[0, 0] += jnp.sum(l * w)


def _filt_body(full_ref, lines_ref, c_ref, filt_ref):
    l = lines_ref[...] + 1e-10
    w = jnp.log(l / (full_ref[...] + 1.0))
    filt_ref[...] = w > c_ref[0, 0]


def _ratios(full_f32, lines_f32):
    f2 = full_f32.reshape(_RROWS, 128)
    l2 = lines_f32.reshape(_RROWS, 128)
    new_full, sl, slw = pl.pallas_call(
        _sums_body,
        grid=(_RGRID,),
        in_specs=[pl.BlockSpec((_RBLK, 128), lambda i: (i, 0))] * 2,
        out_specs=[
            pl.BlockSpec((_RBLK, 128), lambda i: (i, 0)),
            pl.BlockSpec(memory_space=pltpu.SMEM),
            pl.BlockSpec(memory_space=pltpu.SMEM),
        ],
        out_shape=[
            jax.ShapeDtypeStruct((_RROWS, 128), jnp.int32),
            jax.ShapeDtypeStruct((1, 1), jnp.float32),
            jax.ShapeDtypeStruct((1, 1), jnp.float32),
        ],
    )(f2, l2)
    c = slw / sl
    filt = pl.pallas_call(
        _filt_body,
        grid=(_RGRID,),
        in_specs=[pl.BlockSpec((_RBLK, 128), lambda i: (i, 0))] * 2
        + [pl.BlockSpec(memory_space=pltpu.SMEM)],
        out_specs=pl.BlockSpec((_RBLK, 128), lambda i: (i, 0)),
        out_shape=jax.ShapeDtypeStruct((_RROWS, 128), jnp.bool_),
    )(f2, l2, c)
    return new_full, filt


def kernel(img, mask, full, lines):
    r8 = img[:, :, 0]
    g8 = img[:, :, 1]
    b8 = img[:, :, 2]
    idx_lo, idx_hi, mask_p = _bin_index(r8, g8, b8, mask)
    full_f32, lines_f32 = _make_hist_sc()(
        idx_lo.reshape(NPIX), idx_hi.reshape(NPIX), mask_p.reshape(NPIX))
    new_full, filt = _ratios(full_f32, lines_f32)
    return (new_full.reshape(128, 128, 128),
            lines_f32.reshape(128, 128, 128),
            filt.reshape(128, 128, 128))
